# baseline (speedup 1.0000x reference)
import functools

import jax
import jax.numpy as jnp
from jax import lax
from jax.experimental import pallas as pl
from jax.experimental.pallas import tpu as pltpu
from jax.experimental.pallas import tpu_sc as plsc

_NE = 8
_TILE = 512


def _moe_kernel(x_ref, rw_ref, w1_ref, b1_ref, w2_ref, b2_ref,
                out_ref, lt_ref):
    x = x_ref[...]

    logits = jnp.dot(x, rw_ref[...], preferred_element_type=jnp.float32)
    lane = jax.lax.broadcasted_iota(jnp.int32, logits.shape, 1)
    m = jnp.max(logits, axis=1, keepdims=True)
    e = jnp.exp(logits - m)
    probs = e / jnp.sum(e, axis=1, keepdims=True)

    v1 = jnp.max(probs, axis=1, keepdims=True)
    i1 = jnp.min(jnp.where(probs == v1, lane, _NE), axis=1, keepdims=True)
    probs_rest = jnp.where(lane == i1, -1.0, probs)
    v2 = jnp.max(probs_rest, axis=1, keepdims=True)
    s = v1 + v2
    w = v1 / s + v2 / s

    lt_ref[...] = logits.T

    h = jnp.dot(x, w1_ref[...], preferred_element_type=jnp.float32) + b1_ref[...]
    a = jax.nn.gelu(h)
    y = jnp.dot(a, w2_ref[...], preferred_element_type=jnp.float32) + b2_ref[...]
    out_ref[...] = y * w


def _sc_stats_kernel(lt_hbm, parts_hbm, lvmem, pvmem, *, tok_per_w):
    info = plsc.get_sparse_core_info()
    nc = info.num_cores
    wid = lax.axis_index("s") * nc + lax.axis_index("c")
    base = wid * tok_per_w

    for e in range(_NE):
        pltpu.sync_copy(lt_hbm.at[e, pl.ds(base, tok_per_w)], lvmem.at[e])

    nchunk = tok_per_w // 16
    cnt = [jnp.zeros((16,), jnp.float32) for _ in range(_NE)]
    psum = [jnp.zeros((16,), jnp.float32) for _ in range(_NE)]
    for c in range(nchunk):
        l = [lvmem[e, pl.ds(c * 16, 16)] for e in range(_NE)]
        m = l[0]
        for e in range(1, _NE):
            m = jnp.maximum(m, l[e])
        ex = [jnp.exp(l[e] - m) for e in range(_NE)]
        z = ex[0]
        for e in range(1, _NE):
            z = z + ex[e]
        p = [ex[e] / z for e in range(_NE)]
        for e in range(_NE):
            psum[e] = psum[e] + p[e]

        v1 = p[0]
        for e in range(1, _NE):
            v1 = jnp.maximum(v1, p[e])
        seen = jnp.zeros((16,), jnp.float32)
        mask1 = []
        for e in range(_NE):
            eq1 = jnp.where(p[e] == v1, 1.0, 0.0)
            is1 = eq1 * (1.0 - seen)
            seen = seen + is1
            mask1.append(is1)
        p2 = [p[e] - mask1[e] * (p[e] + 1.0) for e in range(_NE)]
        v2 = p2[0]
        for e in range(1, _NE):
            v2 = jnp.maximum(v2, p2[e])
        seen2 = jnp.zeros((16,), jnp.float32)
        for e in range(_NE):
            eq2 = jnp.where(p2[e] == v2, 1.0, 0.0)
            is2 = eq2 * (1.0 - seen2)
            seen2 = seen2 + is2
            cnt[e] = cnt[e] + mask1[e] + is2

    for e in range(_NE):
        pvmem[0, e, :] = cnt[e]
        pvmem[1, e, :] = psum[e]
    pltpu.sync_copy(pvmem, parts_hbm.at[wid])


def kernel(x, router_weights, W1, b1, W2, b2):
    B, S, H = x.shape
    F = W1.shape[1]
    T = B * S
    xs = x.reshape(T, H)
    b1r = b1.reshape(1, F)
    b2r = b2.reshape(1, H)
    grid = T // _TILE

    out, logits_t = pl.pallas_call(
        _moe_kernel,
        grid=(grid,),
        in_specs=[
            pl.BlockSpec((_TILE, H), lambda i: (i, 0)),
            pl.BlockSpec((H, _NE), lambda i: (0, 0)),
            pl.BlockSpec((H, F), lambda i: (0, 0)),
            pl.BlockSpec((1, F), lambda i: (0, 0)),
            pl.BlockSpec((F, H), lambda i: (0, 0)),
            pl.BlockSpec((1, H), lambda i: (0, 0)),
        ],
        out_specs=[
            pl.BlockSpec((_TILE, H), lambda i: (i, 0)),
            pl.BlockSpec((_NE, _TILE), lambda i: (0, i)),
        ],
        out_shape=[
            jax.ShapeDtypeStruct((T, H), jnp.float32),
            jax.ShapeDtypeStruct((_NE, T), jnp.float32),
        ],
    )(xs, router_weights, W1, b1r, W2, b2r)

    info = plsc.get_sparse_core_info()
    nw = info.num_cores * info.num_subcores
    tok_per_w = T // nw
    mesh = plsc.VectorSubcoreMesh(core_axis_name="c", subcore_axis_name="s")

    sc_kernel = functools.partial(
        pl.kernel,
        mesh=mesh,
        out_type=jax.ShapeDtypeStruct((nw, 2, _NE, 16), jnp.float32),
        scratch_types=[
            pltpu.VMEM((_NE, tok_per_w), jnp.float32),
            pltpu.VMEM((2, _NE, 16), jnp.float32),
        ],
    )(functools.partial(_sc_stats_kernel, tok_per_w=tok_per_w))
    parts = sc_kernel(logits_t)

    cnt = jnp.sum(parts[:, 0, :, :], axis=(0, 2))
    psum = jnp.sum(parts[:, 1, :, :], axis=(0, 2))
    density = cnt / T
    proxy = psum / T
    loss = jnp.mean(density * proxy) * (_NE * _NE)

    capacity = max(int(T * 1.25 * 2 / _NE), 4)
    return (out.reshape(B, S, H), loss,
            jnp.asarray(capacity, dtype=jnp.int32))


# speedup vs baseline: 1.7354x; 1.7354x over previous
import functools

import jax
import jax.numpy as jnp
from jax.experimental import pallas as pl
from jax.experimental.pallas import tpu as pltpu

_NE = 8
_LANES = 128
_TILE = 512


def _moe_kernel(x_ref, rw_ref, w1_ref, b1_ref, w2_ref, b2_ref,
                out_ref, loss_ref, acc_ref, *, tokens, grid):
    step = pl.program_id(0)

    @pl.when(step == 0)
    def _init():
        acc_ref[...] = jnp.zeros_like(acc_ref)

    x = x_ref[...]

    logits = jnp.dot(x, rw_ref[...], preferred_element_type=jnp.float32)
    lane = jax.lax.broadcasted_iota(jnp.int32, logits.shape, 1)
    valid = lane < _NE
    logits = jnp.where(valid, logits, -jnp.inf)
    m = jnp.max(logits, axis=1, keepdims=True)
    e = jnp.exp(logits - m)
    probs = e / jnp.sum(e, axis=1, keepdims=True)

    v1 = jnp.max(probs, axis=1, keepdims=True)
    i1 = jnp.min(jnp.where(probs == v1, lane, _LANES), axis=1, keepdims=True)
    mask1 = lane == i1
    probs_rest = jnp.where(mask1, -1.0, probs)
    v2 = jnp.max(probs_rest, axis=1, keepdims=True)
    i2 = jnp.min(jnp.where(probs_rest == v2, lane, _LANES), axis=1, keepdims=True)
    mask2 = lane == i2

    s = v1 + v2
    w = v1 / s + v2 / s

    cnt = jnp.sum((mask1 | mask2).astype(jnp.float32), axis=0, keepdims=True)
    psum = jnp.sum(probs, axis=0, keepdims=True)
    acc_ref[0:1, 0:_NE] += cnt
    acc_ref[1:2, 0:_NE] += psum

    h = jnp.dot(x, w1_ref[...], preferred_element_type=jnp.float32) + b1_ref[...]
    a = jax.nn.gelu(h)
    y = jnp.dot(a, w2_ref[...], preferred_element_type=jnp.float32) + b2_ref[...]
    out_ref[...] = y * w

    @pl.when(step == grid - 1)
    def _finish():
        inv_t = 1.0 / tokens
        density = acc_ref[0:1, 0:_NE] * inv_t
        proxy = acc_ref[1:2, 0:_NE] * inv_t
        loss_ref[0, 0] = jnp.sum(density * proxy) * (_NE * _NE / _NE)


def kernel(x, router_weights, W1, b1, W2, b2):
    B, S, H = x.shape
    F = W1.shape[1]
    T = B * S
    xs = x.reshape(T, H)
    rw_pad = router_weights
    b1r = b1.reshape(1, F)
    b2r = b2.reshape(1, H)
    grid = T // _TILE

    out, loss = pl.pallas_call(
        functools.partial(_moe_kernel, tokens=float(T), grid=grid),
        grid=(grid,),
        in_specs=[
            pl.BlockSpec((_TILE, H), lambda i: (i, 0)),
            pl.BlockSpec((H, _NE), lambda i: (0, 0)),
            pl.BlockSpec((H, F), lambda i: (0, 0)),
            pl.BlockSpec((1, F), lambda i: (0, 0)),
            pl.BlockSpec((F, H), lambda i: (0, 0)),
            pl.BlockSpec((1, H), lambda i: (0, 0)),
        ],
        out_specs=[
            pl.BlockSpec((_TILE, H), lambda i: (i, 0)),
            pl.BlockSpec(memory_space=pltpu.SMEM, block_shape=(1, 1),
                         index_map=lambda i: (0, 0)),
        ],
        out_shape=[
            jax.ShapeDtypeStruct((T, H), jnp.float32),
            jax.ShapeDtypeStruct((1, 1), jnp.float32),
        ],
        scratch_shapes=[pltpu.VMEM((8, _LANES), jnp.float32)],
    )(xs, rw_pad, W1, b1r, W2, b2r)

    capacity = max(int(T * 1.25 * 2 / _NE), 4)
    return (out.reshape(B, S, H), loss[0, 0],
            jnp.asarray(capacity, dtype=jnp.int32))
